# Initial kernel scaffold; baseline (speedup 1.0000x reference)
import jax
import jax.numpy as jnp
from jax.experimental import pallas as pl


def kernel(x, edge_index, edge_weigth, batch, W1, b1, g1, beta1, W2, b2, g2, beta2, fc1_w, fc1_b, fc2_w, fc2_b, fc3_w, fc3_b):
    raise NotImplementedError("write your pallas kernel here")


# speedup vs baseline: 6.1381x; 6.1381x over previous
import functools

import jax
import jax.numpy as jnp
from jax import lax
from jax.experimental import pallas as pl
from jax.experimental.pallas import tpu as pltpu
from jax.experimental.pallas import tpu_sc as plsc

N_NODES = 10000
N_GRAPHS = 64
D_FEAT = 256
EPS = 1e-5

NC = 2
NS = 16
NW = NC * NS
N_PAD = 10240
ROWS_PER_TILE = N_PAD // NS
E = 160000
CH = 128
NCH = 40
E_PER_W = CH * NCH
E_PAD = NW * E_PER_W


def _sc_aggregate(h, srcp, dstp, wp, zrows):
  mesh = plsc.VectorSubcoreMesh(
      core_axis_name="c", subcore_axis_name="s", num_cores=NC,
      num_subcores=NS)

  @functools.partial(
      pl.kernel,
      out_type=jax.ShapeDtypeStruct((NC, N_PAD, 32), jnp.float32),
      mesh=mesh,
      compiler_params=pltpu.CompilerParams(use_tc_tiling_on_sc=False),
      scratch_types=[
          pltpu.VMEM((NCH, CH), jnp.int32),
          pltpu.VMEM((NCH, CH), jnp.int32),
          pltpu.VMEM((NCH, CH), jnp.float32),
          pltpu.VMEM((CH, 32), jnp.float32),
          pltpu.VMEM_SHARED((N_PAD, 32), jnp.float32),
          pltpu.SemaphoreType.DMA,
      ],
  )
  def body(h_hbm, src_hbm, dst_hbm, w_hbm, z_hbm, out_hbm,
           src_v, dst_v, w_v, msg_v, acc_sh, sem):
    cid = lax.axis_index("c")
    sid = lax.axis_index("s")
    wid = sid * NC + cid

    pltpu.sync_copy(z_hbm, acc_sh.at[pl.ds(sid * ROWS_PER_TILE,
                                           ROWS_PER_TILE)])
    pltpu.sync_copy(src_hbm.at[wid], src_v)
    pltpu.sync_copy(dst_hbm.at[wid], dst_v)
    pltpu.sync_copy(w_hbm.at[wid], w_v)
    plsc.subcore_barrier()

    def chunk(j, carry):
      pltpu.async_copy(h_hbm.at[src_v.at[j]], msg_v, sem).wait()

      for g in range(CH // 16):
        wg = w_v[j, pl.ds(g * 16, 16)]
        for t in range(16):
          e = g * 16 + t
          wv = jnp.full((16,), wg[t], jnp.float32)
          msg_v[e, pl.ds(0, 16)] = msg_v[e, pl.ds(0, 16)] * wv
          msg_v[e, pl.ds(16, 16)] = msg_v[e, pl.ds(16, 16)] * wv

      pltpu.sync_copy(msg_v, acc_sh.at[dst_v.at[j]], add=True)
      return carry

    lax.fori_loop(0, NCH, chunk, 0)
    plsc.subcore_barrier()

    pltpu.sync_copy(
        acc_sh.at[pl.ds(sid * ROWS_PER_TILE, ROWS_PER_TILE)],
        out_hbm.at[cid, pl.ds(sid * ROWS_PER_TILE, ROWS_PER_TILE)])

  return body(h, srcp, dstp, wp, zrows)


def _mm_body(x_ref, w_ref, o_ref):
  o_ref[...] = jnp.dot(x_ref[...], w_ref[...],
                       preferred_element_type=jnp.float32)


def _matmul_x_w1(x, w1):
  return pl.pallas_call(
      _mm_body,
      grid=(10,),
      in_specs=[
          pl.BlockSpec((1000, D_FEAT), lambda i: (i, 0)),
          pl.BlockSpec((D_FEAT, 32), lambda i: (0, 0)),
      ],
      out_specs=pl.BlockSpec((1000, 32), lambda i: (i, 0)),
      out_shape=jax.ShapeDtypeStruct((N_NODES, 32), jnp.float32),
  )(x, w1)


def _leaky(v):
  return jnp.where(v >= 0, v, 0.01 * v)


def _bn_leaky_body(p0_ref, p1_ref, b_ref, g_ref, beta_ref, o_ref):
  s = p0_ref[...] + p1_ref[...] + b_ref[...]
  m = jnp.mean(s, axis=0, keepdims=True)
  v = jnp.mean((s - m) ** 2, axis=0, keepdims=True)
  h = (s - m) / jnp.sqrt(v + EPS) * g_ref[...] + beta_ref[...]
  o_ref[...] = _leaky(h)


def _bn_leaky(p0, p1, b, g, beta):
  return pl.pallas_call(
      _bn_leaky_body,
      out_shape=jax.ShapeDtypeStruct((N_NODES, 32), jnp.float32),
  )(p0, p1, b.reshape(1, 32), g.reshape(1, 32), beta.reshape(1, 32))


def _tail_body(p0_ref, p1_ref, w2_ref, b2_ref, g2_ref, beta2_ref,
               batch_ref, fc1w_ref, fc1b_ref, fc2w_ref, fc2b_ref,
               fc3w_ref, fc3b_ref, o_ref):
  s = p0_ref[...] + p1_ref[...]
  z = jnp.dot(s, w2_ref[...], preferred_element_type=jnp.float32)
  z = z + b2_ref[...]
  m = jnp.mean(z, axis=0, keepdims=True)
  v = jnp.mean((z - m) ** 2, axis=0, keepdims=True)
  h = _leaky((z - m) / jnp.sqrt(v + EPS) * g2_ref[...] + beta2_ref[...])
  gids = lax.broadcasted_iota(jnp.int32, (N_GRAPHS, N_NODES), 0)
  onehot = (gids == batch_ref[...]).astype(jnp.float32)
  pooled = jnp.dot(onehot, h, preferred_element_type=jnp.float32)
  o1 = _leaky(jnp.dot(pooled, fc1w_ref[...],
                      preferred_element_type=jnp.float32) + fc1b_ref[...])
  o2 = _leaky(jnp.dot(o1, fc2w_ref[...],
                      preferred_element_type=jnp.float32) + fc2b_ref[...])
  o3 = _leaky(jnp.dot(o2, fc3w_ref[...],
                      preferred_element_type=jnp.float32) + fc3b_ref[...])
  o_ref[...] = o3


def _tail(p0, p1, w2, b2, g2, beta2, batch, fc1_w, fc1_b, fc2_w, fc2_b,
          fc3_w, fc3_b):
  return pl.pallas_call(
      _tail_body,
      out_shape=jax.ShapeDtypeStruct((N_GRAPHS, 2), jnp.float32),
  )(p0, p1, w2, b2.reshape(1, 64), g2.reshape(1, 64),
    beta2.reshape(1, 64), batch.reshape(1, N_NODES),
    fc1_w.T, fc1_b.reshape(1, 32), fc2_w.T, fc2_b.reshape(1, 16),
    fc3_w.T, fc3_b.reshape(1, 2))


def kernel(x, edge_index, edge_weigth, batch, W1, b1, g1, beta1,
           W2, b2, g2, beta2, fc1_w, fc1_b, fc2_w, fc2_b, fc3_w, fc3_b):
  pad = E_PAD - E
  srcp = jnp.concatenate(
      [edge_index[0], jnp.zeros((pad,), jnp.int32)]).reshape(NW, NCH, CH)
  dstp = jnp.concatenate(
      [edge_index[1], jnp.zeros((pad,), jnp.int32)]).reshape(NW, NCH, CH)
  wp = jnp.concatenate(
      [edge_weigth, jnp.zeros((pad,), jnp.float32)]).reshape(NW, NCH, CH)
  zrows = jnp.zeros((ROWS_PER_TILE, 32), jnp.float32)

  h0 = _matmul_x_w1(x, W1)
  parts1 = _sc_aggregate(h0, srcp, dstp, wp, zrows)
  h1 = _bn_leaky(parts1[0, :N_NODES], parts1[1, :N_NODES], b1, g1, beta1)

  parts2 = _sc_aggregate(h1, srcp, dstp, wp, zrows)
  return _tail(parts2[0, :N_NODES], parts2[1, :N_NODES], W2, b2, g2,
               beta2, batch, fc1_w, fc1_b, fc2_w, fc2_b, fc3_w, fc3_b)


# speedup vs baseline: 6.7485x; 1.0994x over previous
import functools

import jax
import jax.numpy as jnp
from jax import lax
from jax.experimental import pallas as pl
from jax.experimental.pallas import tpu as pltpu
from jax.experimental.pallas import tpu_sc as plsc

N_NODES = 10000
N_GRAPHS = 64
D_FEAT = 256
EPS = 1e-5

NC = 2
NS = 16
NW = NC * NS
N_PAD = 10240
ROWS_PER_TILE = N_PAD // NS
E = 160000
CH = 128
NCH = 40
E_PER_W = CH * NCH
E_PAD = NW * E_PER_W


def _sc_aggregate(h, srcp, dstp, wp, zrows):
  mesh = plsc.VectorSubcoreMesh(
      core_axis_name="c", subcore_axis_name="s", num_cores=NC,
      num_subcores=NS)

  @functools.partial(
      pl.kernel,
      out_type=jax.ShapeDtypeStruct((NC, N_PAD, 32), jnp.float32),
      mesh=mesh,
      compiler_params=pltpu.CompilerParams(use_tc_tiling_on_sc=False),
      scratch_types=[
          pltpu.VMEM((NCH, CH), jnp.int32),
          pltpu.VMEM((NCH, CH), jnp.int32),
          pltpu.VMEM((NCH, CH), jnp.float32),
          pltpu.VMEM((CH, 32), jnp.float32),
          pltpu.VMEM((CH, 32), jnp.float32),
          pltpu.VMEM_SHARED((N_PAD, 32), jnp.float32),
          pltpu.SemaphoreType.DMA,
          pltpu.SemaphoreType.DMA,
          pltpu.SemaphoreType.DMA,
          pltpu.SemaphoreType.DMA,
      ],
  )
  def body(h_hbm, src_hbm, dst_hbm, w_hbm, z_hbm, out_hbm,
           src_v, dst_v, w_v, msg_a, msg_b, acc_sh,
           sem_ga, sem_gb, sem_sa, sem_sb):
    cid = lax.axis_index("c")
    sid = lax.axis_index("s")
    wid = sid * NC + cid

    pltpu.sync_copy(z_hbm, acc_sh.at[pl.ds(sid * ROWS_PER_TILE,
                                           ROWS_PER_TILE)])
    pltpu.sync_copy(src_hbm.at[wid], src_v)
    pltpu.sync_copy(dst_hbm.at[wid], dst_v)
    pltpu.sync_copy(w_hbm.at[wid], w_v)
    plsc.subcore_barrier()

    def scale(j, msg_v):
      for g in range(CH // 16):
        wg = w_v[j, pl.ds(g * 16, 16)]
        for t in range(16):
          e = g * 16 + t
          wv = jnp.full((16,), wg[t], jnp.float32)
          msg_v[e, pl.ds(0, 16)] = msg_v[e, pl.ds(0, 16)] * wv
          msg_v[e, pl.ds(16, 16)] = msg_v[e, pl.ds(16, 16)] * wv

    def two_chunks(i, carry):
      j0 = 2 * i
      j1 = 2 * i + 1
      g_a = pltpu.async_copy(h_hbm.at[src_v.at[j0]], msg_a, sem_ga)
      g_b = pltpu.async_copy(h_hbm.at[src_v.at[j1]], msg_b, sem_gb)
      g_a.wait()
      scale(j0, msg_a)
      s_a = pltpu.async_copy(msg_a, acc_sh.at[dst_v.at[j0]], sem_sa,
                             add=True)
      g_b.wait()
      scale(j1, msg_b)
      s_b = pltpu.async_copy(msg_b, acc_sh.at[dst_v.at[j1]], sem_sb,
                             add=True)
      s_a.wait()
      s_b.wait()
      return carry

    lax.fori_loop(0, NCH // 2, two_chunks, 0)
    plsc.subcore_barrier()

    pltpu.sync_copy(
        acc_sh.at[pl.ds(sid * ROWS_PER_TILE, ROWS_PER_TILE)],
        out_hbm.at[cid, pl.ds(sid * ROWS_PER_TILE, ROWS_PER_TILE)])

  return body(h, srcp, dstp, wp, zrows)


def _mm_body(x_ref, w_ref, o_ref):
  o_ref[...] = jnp.dot(x_ref[...], w_ref[...],
                       preferred_element_type=jnp.float32)


def _matmul_x_w1(x, w1):
  return pl.pallas_call(
      _mm_body,
      grid=(10,),
      in_specs=[
          pl.BlockSpec((1000, D_FEAT), lambda i: (i, 0)),
          pl.BlockSpec((D_FEAT, 32), lambda i: (0, 0)),
      ],
      out_specs=pl.BlockSpec((1000, 32), lambda i: (i, 0)),
      out_shape=jax.ShapeDtypeStruct((N_NODES, 32), jnp.float32),
  )(x, w1)


def _leaky(v):
  return jnp.where(v >= 0, v, 0.01 * v)


def _bn_leaky_body(p0_ref, p1_ref, b_ref, g_ref, beta_ref, o_ref):
  s = p0_ref[...] + p1_ref[...] + b_ref[...]
  m = jnp.mean(s, axis=0, keepdims=True)
  v = jnp.mean((s - m) ** 2, axis=0, keepdims=True)
  h = (s - m) / jnp.sqrt(v + EPS) * g_ref[...] + beta_ref[...]
  o_ref[...] = _leaky(h)


def _bn_leaky(p0, p1, b, g, beta):
  return pl.pallas_call(
      _bn_leaky_body,
      out_shape=jax.ShapeDtypeStruct((N_NODES, 32), jnp.float32),
  )(p0, p1, b.reshape(1, 32), g.reshape(1, 32), beta.reshape(1, 32))


def _tail_body(p0_ref, p1_ref, w2_ref, b2_ref, g2_ref, beta2_ref,
               batch_ref, fc1w_ref, fc1b_ref, fc2w_ref, fc2b_ref,
               fc3w_ref, fc3b_ref, o_ref):
  s = p0_ref[...] + p1_ref[...]
  z = jnp.dot(s, w2_ref[...], preferred_element_type=jnp.float32)
  z = z + b2_ref[...]
  m = jnp.mean(z, axis=0, keepdims=True)
  v = jnp.mean((z - m) ** 2, axis=0, keepdims=True)
  h = _leaky((z - m) / jnp.sqrt(v + EPS) * g2_ref[...] + beta2_ref[...])
  gids = lax.broadcasted_iota(jnp.int32, (N_GRAPHS, N_NODES), 0)
  onehot = (gids == batch_ref[...]).astype(jnp.float32)
  pooled = jnp.dot(onehot, h, preferred_element_type=jnp.float32)
  o1 = _leaky(jnp.dot(pooled, fc1w_ref[...],
                      preferred_element_type=jnp.float32) + fc1b_ref[...])
  o2 = _leaky(jnp.dot(o1, fc2w_ref[...],
                      preferred_element_type=jnp.float32) + fc2b_ref[...])
  o3 = _leaky(jnp.dot(o2, fc3w_ref[...],
                      preferred_element_type=jnp.float32) + fc3b_ref[...])
  o_ref[...] = o3


def _tail(p0, p1, w2, b2, g2, beta2, batch, fc1_w, fc1_b, fc2_w, fc2_b,
          fc3_w, fc3_b):
  return pl.pallas_call(
      _tail_body,
      out_shape=jax.ShapeDtypeStruct((N_GRAPHS, 2), jnp.float32),
  )(p0, p1, w2, b2.reshape(1, 64), g2.reshape(1, 64),
    beta2.reshape(1, 64), batch.reshape(1, N_NODES),
    fc1_w.T, fc1_b.reshape(1, 32), fc2_w.T, fc2_b.reshape(1, 16),
    fc3_w.T, fc3_b.reshape(1, 2))


def kernel(x, edge_index, edge_weigth, batch, W1, b1, g1, beta1,
           W2, b2, g2, beta2, fc1_w, fc1_b, fc2_w, fc2_b, fc3_w, fc3_b):
  pad = E_PAD - E
  srcp = jnp.concatenate(
      [edge_index[0], jnp.zeros((pad,), jnp.int32)]).reshape(NW, NCH, CH)
  dstp = jnp.concatenate(
      [edge_index[1], jnp.zeros((pad,), jnp.int32)]).reshape(NW, NCH, CH)
  wp = jnp.concatenate(
      [edge_weigth, jnp.zeros((pad,), jnp.float32)]).reshape(NW, NCH, CH)
  zrows = jnp.zeros((ROWS_PER_TILE, 32), jnp.float32)

  h0 = _matmul_x_w1(x, W1)
  parts1 = _sc_aggregate(h0, srcp, dstp, wp, zrows)
  h1 = _bn_leaky(parts1[0, :N_NODES], parts1[1, :N_NODES], b1, g1, beta1)

  parts2 = _sc_aggregate(h1, srcp, dstp, wp, zrows)
  return _tail(parts2[0, :N_NODES], parts2[1, :N_NODES], W2, b2, g2,
               beta2, batch, fc1_w, fc1_b, fc2_w, fc2_b, fc3_w, fc3_b)


# speedup vs baseline: 10.7536x; 1.5935x over previous
import functools

import jax
import jax.numpy as jnp
from jax import lax
from jax.experimental import pallas as pl
from jax.experimental.pallas import tpu as pltpu
from jax.experimental.pallas import tpu_sc as plsc

N_NODES = 10000
N_GRAPHS = 64
D_FEAT = 256
EPS = 1e-5

NC = 2
NS = 16
NW = NC * NS
N_PAD = 10240
ROWS_PER_TILE = N_PAD // NS
E = 160000
CH = 512
NCH = 10
E_PER_W = CH * NCH
E_PAD = NW * E_PER_W


def _sc_aggregate(h, srcp, dstp, wp, zrows):
  mesh = plsc.VectorSubcoreMesh(
      core_axis_name="c", subcore_axis_name="s", num_cores=NC,
      num_subcores=NS)

  @functools.partial(
      pl.kernel,
      out_type=jax.ShapeDtypeStruct((NC, N_PAD, 32), jnp.float32),
      mesh=mesh,
      compiler_params=pltpu.CompilerParams(use_tc_tiling_on_sc=False),
      scratch_types=[
          pltpu.VMEM((NCH, CH), jnp.int32),
          pltpu.VMEM((NCH, CH), jnp.int32),
          pltpu.VMEM((NCH, CH), jnp.float32),
          pltpu.VMEM((CH, 32), jnp.float32),
          pltpu.VMEM((CH, 32), jnp.float32),
          pltpu.VMEM_SHARED((N_NODES, 32), jnp.float32),
          pltpu.VMEM_SHARED((N_PAD, 32), jnp.float32),
          pltpu.SemaphoreType.DMA,
          pltpu.SemaphoreType.DMA,
          pltpu.SemaphoreType.DMA,
          pltpu.SemaphoreType.DMA,
      ],
  )
  def body(h_hbm, src_hbm, dst_hbm, w_hbm, z_hbm, out_hbm,
           src_v, dst_v, w_v, msg_a, msg_b, h_sh, acc_sh,
           sem_ga, sem_gb, sem_sa, sem_sb):
    cid = lax.axis_index("c")
    sid = lax.axis_index("s")
    wid = sid * NC + cid

    pltpu.sync_copy(z_hbm, acc_sh.at[pl.ds(sid * ROWS_PER_TILE,
                                           ROWS_PER_TILE)])
    pltpu.sync_copy(h_hbm.at[pl.ds(sid * (N_NODES // NS), N_NODES // NS)],
                    h_sh.at[pl.ds(sid * (N_NODES // NS), N_NODES // NS)])
    pltpu.sync_copy(src_hbm.at[wid], src_v)
    pltpu.sync_copy(dst_hbm.at[wid], dst_v)
    pltpu.sync_copy(w_hbm.at[wid], w_v)
    plsc.subcore_barrier()

    def scale(j, msg_v):
      def group(g, carry):
        base = g * 16
        wg = w_v[j, pl.ds(base, 16)]
        for t in range(16):
          e = base + t
          wv = jnp.full((16,), wg[t], jnp.float32)
          msg_v[e, pl.ds(0, 16)] = msg_v[e, pl.ds(0, 16)] * wv
          msg_v[e, pl.ds(16, 16)] = msg_v[e, pl.ds(16, 16)] * wv
        return carry

      lax.fori_loop(0, CH // 16, group, 0)

    def two_chunks(i, carry):
      j0 = 2 * i
      j1 = 2 * i + 1
      g_a = pltpu.async_copy(h_sh.at[src_v.at[j0]], msg_a, sem_ga)
      g_b = pltpu.async_copy(h_sh.at[src_v.at[j1]], msg_b, sem_gb)
      g_a.wait()
      scale(j0, msg_a)
      s_a = pltpu.async_copy(msg_a, acc_sh.at[dst_v.at[j0]], sem_sa,
                             add=True)
      g_b.wait()
      scale(j1, msg_b)
      s_b = pltpu.async_copy(msg_b, acc_sh.at[dst_v.at[j1]], sem_sb,
                             add=True)
      s_a.wait()
      s_b.wait()
      return carry

    lax.fori_loop(0, NCH // 2, two_chunks, 0)
    plsc.subcore_barrier()

    pltpu.sync_copy(
        acc_sh.at[pl.ds(sid * ROWS_PER_TILE, ROWS_PER_TILE)],
        out_hbm.at[cid, pl.ds(sid * ROWS_PER_TILE, ROWS_PER_TILE)])

  return body(h, srcp, dstp, wp, zrows)


def _mm_body(x_ref, w_ref, o_ref):
  o_ref[...] = jnp.dot(x_ref[...], w_ref[...],
                       preferred_element_type=jnp.float32)


def _matmul_x_w1(x, w1):
  return pl.pallas_call(
      _mm_body,
      grid=(10,),
      in_specs=[
          pl.BlockSpec((1000, D_FEAT), lambda i: (i, 0)),
          pl.BlockSpec((D_FEAT, 32), lambda i: (0, 0)),
      ],
      out_specs=pl.BlockSpec((1000, 32), lambda i: (i, 0)),
      out_shape=jax.ShapeDtypeStruct((N_NODES, 32), jnp.float32),
  )(x, w1)


def _leaky(v):
  return jnp.where(v >= 0, v, 0.01 * v)


def _bn_leaky_body(p0_ref, p1_ref, b_ref, g_ref, beta_ref, o_ref):
  s = p0_ref[...] + p1_ref[...] + b_ref[...]
  m = jnp.mean(s, axis=0, keepdims=True)
  v = jnp.mean((s - m) ** 2, axis=0, keepdims=True)
  h = (s - m) / jnp.sqrt(v + EPS) * g_ref[...] + beta_ref[...]
  o_ref[...] = _leaky(h)


def _bn_leaky(p0, p1, b, g, beta):
  return pl.pallas_call(
      _bn_leaky_body,
      out_shape=jax.ShapeDtypeStruct((N_NODES, 32), jnp.float32),
  )(p0, p1, b.reshape(1, 32), g.reshape(1, 32), beta.reshape(1, 32))


def _tail_body(p0_ref, p1_ref, w2_ref, b2_ref, g2_ref, beta2_ref,
               batch_ref, fc1w_ref, fc1b_ref, fc2w_ref, fc2b_ref,
               fc3w_ref, fc3b_ref, o_ref):
  s = p0_ref[...] + p1_ref[...]
  z = jnp.dot(s, w2_ref[...], preferred_element_type=jnp.float32)
  z = z + b2_ref[...]
  m = jnp.mean(z, axis=0, keepdims=True)
  v = jnp.mean((z - m) ** 2, axis=0, keepdims=True)
  h = _leaky((z - m) / jnp.sqrt(v + EPS) * g2_ref[...] + beta2_ref[...])
  gids = lax.broadcasted_iota(jnp.int32, (N_GRAPHS, N_NODES), 0)
  onehot = (gids == batch_ref[...]).astype(jnp.float32)
  pooled = jnp.dot(onehot, h, preferred_element_type=jnp.float32)
  o1 = _leaky(jnp.dot(pooled, fc1w_ref[...],
                      preferred_element_type=jnp.float32) + fc1b_ref[...])
  o2 = _leaky(jnp.dot(o1, fc2w_ref[...],
                      preferred_element_type=jnp.float32) + fc2b_ref[...])
  o3 = _leaky(jnp.dot(o2, fc3w_ref[...],
                      preferred_element_type=jnp.float32) + fc3b_ref[...])
  o_ref[...] = o3


def _tail(p0, p1, w2, b2, g2, beta2, batch, fc1_w, fc1_b, fc2_w, fc2_b,
          fc3_w, fc3_b):
  return pl.pallas_call(
      _tail_body,
      out_shape=jax.ShapeDtypeStruct((N_GRAPHS, 2), jnp.float32),
  )(p0, p1, w2, b2.reshape(1, 64), g2.reshape(1, 64),
    beta2.reshape(1, 64), batch.reshape(1, N_NODES),
    fc1_w.T, fc1_b.reshape(1, 32), fc2_w.T, fc2_b.reshape(1, 16),
    fc3_w.T, fc3_b.reshape(1, 2))


def kernel(x, edge_index, edge_weigth, batch, W1, b1, g1, beta1,
           W2, b2, g2, beta2, fc1_w, fc1_b, fc2_w, fc2_b, fc3_w, fc3_b):
  pad = E_PAD - E
  srcp = jnp.concatenate(
      [edge_index[0], jnp.zeros((pad,), jnp.int32)]).reshape(NW, NCH, CH)
  dstp = jnp.concatenate(
      [edge_index[1], jnp.zeros((pad,), jnp.int32)]).reshape(NW, NCH, CH)
  wp = jnp.concatenate(
      [edge_weigth, jnp.zeros((pad,), jnp.float32)]).reshape(NW, NCH, CH)
  zrows = jnp.zeros((ROWS_PER_TILE, 32), jnp.float32)

  h0 = _matmul_x_w1(x, W1)
  parts1 = _sc_aggregate(h0, srcp, dstp, wp, zrows)
  h1 = _bn_leaky(parts1[0, :N_NODES], parts1[1, :N_NODES], b1, g1, beta1)

  parts2 = _sc_aggregate(h1, srcp, dstp, wp, zrows)
  return _tail(parts2[0, :N_NODES], parts2[1, :N_NODES], W2, b2, g2,
               beta2, batch, fc1_w, fc1_b, fc2_w, fc2_b, fc3_w, fc3_b)


# speedup vs baseline: 12.2100x; 1.1354x over previous
import functools

import jax
import jax.numpy as jnp
from jax import lax
from jax.experimental import pallas as pl
from jax.experimental.pallas import tpu as pltpu
from jax.experimental.pallas import tpu_sc as plsc

N_NODES = 10000
N_GRAPHS = 64
D_FEAT = 256
EPS = 1e-5

NC = 2
NS = 16
NW = NC * NS
N_PAD = 10240
ROWS_PER_TILE = N_PAD // NS
E = 160000
CH = 512
NCH = 10
E_PER_W = CH * NCH
E_PAD = NW * E_PER_W


def _rsqrt16(x):
  i = plsc.bitcast(x, jnp.int32)
  seed = jnp.full((16,), 0x5F3759DF, jnp.int32) - (i >> 1)
  y = plsc.bitcast(seed, jnp.float32)
  for _ in range(4):
    y = y * (1.5 - 0.5 * x * y * y)
  return y


def _sc_aggregate(h_or_parts, srcp, dstp, wp, zrows, bnc=None):
  fuse_bn = bnc is not None
  mesh = plsc.VectorSubcoreMesh(
      core_axis_name="c", subcore_axis_name="s", num_cores=NC,
      num_subcores=NS)
  NR = N_NODES // NS

  scratch = [
      pltpu.VMEM((NCH, CH), jnp.int32),
      pltpu.VMEM((NCH, CH), jnp.int32),
      pltpu.VMEM((NCH, CH), jnp.float32),
      pltpu.VMEM((CH, 32), jnp.float32),
      pltpu.VMEM((CH, 32), jnp.float32),
      pltpu.VMEM_SHARED((N_NODES, 32), jnp.float32),
      pltpu.VMEM_SHARED((N_PAD, 32), jnp.float32),
      pltpu.SemaphoreType.DMA,
      pltpu.SemaphoreType.DMA,
      pltpu.SemaphoreType.DMA,
      pltpu.SemaphoreType.DMA,
  ]
  if fuse_bn:
    scratch += [
        pltpu.VMEM((NR, 32), jnp.float32),
        pltpu.VMEM((NR, 32), jnp.float32),
        pltpu.VMEM((3, 2, 16), jnp.float32),
        pltpu.VMEM((4, 16), jnp.float32),
        pltpu.VMEM((NS, 4, 16), jnp.float32),
        pltpu.VMEM_SHARED((NS, 4, 16), jnp.float32),
    ]

  def prologue_plain(h_hbm, h_sh, sid, extra):
    pltpu.sync_copy(h_hbm.at[pl.ds(sid * NR, NR)],
                    h_sh.at[pl.ds(sid * NR, NR)])

  def prologue_bn(p_hbm, bnc_hbm, h_sh, sid, extra):
    p0_v, p1_v, bnc_v, st_v, allst_v, st_sh = extra
    pltpu.sync_copy(p_hbm.at[0, pl.ds(sid * NR, NR)], p0_v)
    pltpu.sync_copy(p_hbm.at[1, pl.ds(sid * NR, NR)], p1_v)
    pltpu.sync_copy(bnc_hbm, bnc_v)
    bias = [bnc_v[0, 0], bnc_v[0, 1]]
    zero = jnp.zeros((16,), jnp.float32)

    def sum_rows(r, carry):
      cs0, cs1, cq0, cq1 = carry
      for u in range(5):
        row = r * 5 + u
        v0 = p0_v[row, pl.ds(0, 16)] + p1_v[row, pl.ds(0, 16)] + bias[0]
        v1 = p0_v[row, pl.ds(16, 16)] + p1_v[row, pl.ds(16, 16)] + bias[1]
        p0_v[row, pl.ds(0, 16)] = v0
        p0_v[row, pl.ds(16, 16)] = v1
        cs0 = cs0 + v0
        cs1 = cs1 + v1
        cq0 = cq0 + v0 * v0
        cq1 = cq1 + v1 * v1
      return cs0, cs1, cq0, cq1

    cs0, cs1, cq0, cq1 = lax.fori_loop(0, NR // 5, sum_rows,
                                       (zero, zero, zero, zero))
    st_v[0] = cs0
    st_v[1] = cs1
    st_v[2] = cq0
    st_v[3] = cq1
    pltpu.sync_copy(st_v, st_sh.at[sid])
    plsc.subcore_barrier()
    pltpu.sync_copy(st_sh, allst_v)
    ts = [zero, zero, zero, zero]
    for t in range(NS):
      for k in range(4):
        ts[k] = ts[k] + allst_v[t, k]
    scale_n = jnp.float32(1.0 / N_NODES)
    ab = []
    for hf in range(2):
      mean = ts[hf] * scale_n
      var = ts[2 + hf] * scale_n - mean * mean
      inv = _rsqrt16(var + EPS)
      a = inv * bnc_v[1, hf]
      c = bnc_v[2, hf] - mean * a
      ab.append((a, c))

    def norm_rows(r, carry):
      for u in range(5):
        row = r * 5 + u
        for hf in range(2):
          a, c = ab[hf]
          v = p0_v[row, pl.ds(16 * hf, 16)] * a + c
          p0_v[row, pl.ds(16 * hf, 16)] = jnp.where(v >= 0, v, 0.01 * v)
      return carry

    lax.fori_loop(0, NR // 5, norm_rows, 0)
    pltpu.sync_copy(p0_v, h_sh.at[pl.ds(sid * NR, NR)])

  @functools.partial(
      pl.kernel,
      out_type=jax.ShapeDtypeStruct((NC, N_PAD, 32), jnp.float32),
      mesh=mesh,
      compiler_params=pltpu.CompilerParams(use_tc_tiling_on_sc=False,
                                          needs_layout_passes=False),
      scratch_types=scratch,
  )
  def body(h_hbm, src_hbm, dst_hbm, w_hbm, z_hbm, *rest):
    if fuse_bn:
      bnc_hbm, out_hbm = rest[0], rest[1]
      (src_v, dst_v, w_v, msg_a, msg_b, h_sh, acc_sh,
       sem_ga, sem_gb, sem_sa, sem_sb) = rest[2:13]
      extra = rest[13:]
    else:
      bnc_hbm = None
      out_hbm = rest[0]
      (src_v, dst_v, w_v, msg_a, msg_b, h_sh, acc_sh,
       sem_ga, sem_gb, sem_sa, sem_sb) = rest[1:12]
      extra = ()
    cid = lax.axis_index("c")
    sid = lax.axis_index("s")
    wid = sid * NC + cid

    pltpu.sync_copy(z_hbm, acc_sh.at[pl.ds(sid * ROWS_PER_TILE,
                                           ROWS_PER_TILE)])
    pltpu.sync_copy(src_hbm.at[wid], src_v)
    pltpu.sync_copy(dst_hbm.at[wid], dst_v)
    pltpu.sync_copy(w_hbm.at[wid], w_v)
    if fuse_bn:
      prologue_bn(h_hbm, bnc_hbm, h_sh, sid, extra)
    else:
      prologue_plain(h_hbm, h_sh, sid, extra)
    plsc.subcore_barrier()

    def scale(j, msg_v):
      def group(g, carry):
        base = g * 16
        wg = w_v[j, pl.ds(base, 16)]
        for t in range(16):
          e = base + t
          wv = jnp.full((16,), wg[t], jnp.float32)
          msg_v[e, pl.ds(0, 16)] = msg_v[e, pl.ds(0, 16)] * wv
          msg_v[e, pl.ds(16, 16)] = msg_v[e, pl.ds(16, 16)] * wv
        return carry

      lax.fori_loop(0, CH // 16, group, 0)

    def two_chunks(i, carry):
      j0 = 2 * i
      j1 = 2 * i + 1
      g_a = pltpu.async_copy(h_sh.at[src_v.at[j0]], msg_a, sem_ga)
      g_b = pltpu.async_copy(h_sh.at[src_v.at[j1]], msg_b, sem_gb)
      g_a.wait()
      scale(j0, msg_a)
      s_a = pltpu.async_copy(msg_a, acc_sh.at[dst_v.at[j0]], sem_sa,
                             add=True)
      g_b.wait()
      scale(j1, msg_b)
      s_b = pltpu.async_copy(msg_b, acc_sh.at[dst_v.at[j1]], sem_sb,
                             add=True)
      s_a.wait()
      s_b.wait()
      return carry

    lax.fori_loop(0, NCH // 2, two_chunks, 0)
    plsc.subcore_barrier()

    pltpu.sync_copy(
        acc_sh.at[pl.ds(sid * ROWS_PER_TILE, ROWS_PER_TILE)],
        out_hbm.at[cid, pl.ds(sid * ROWS_PER_TILE, ROWS_PER_TILE)])

  if fuse_bn:
    return body(h_or_parts, srcp, dstp, wp, zrows, bnc)
  return body(h_or_parts, srcp, dstp, wp, zrows)


def _mm_body(x_ref, w_ref, o_ref):
  o_ref[...] = jnp.dot(x_ref[...], w_ref[...],
                       preferred_element_type=jnp.float32)


def _matmul_x_w1(x, w1):
  return pl.pallas_call(
      _mm_body,
      grid=(10,),
      in_specs=[
          pl.BlockSpec((1000, D_FEAT), lambda i: (i, 0)),
          pl.BlockSpec((D_FEAT, 32), lambda i: (0, 0)),
      ],
      out_specs=pl.BlockSpec((1000, 32), lambda i: (i, 0)),
      out_shape=jax.ShapeDtypeStruct((N_NODES, 32), jnp.float32),
  )(x, w1)


def _leaky(v):
  return jnp.where(v >= 0, v, 0.01 * v)


def _bn_leaky_body(p0_ref, p1_ref, b_ref, g_ref, beta_ref, o_ref):
  s = p0_ref[...] + p1_ref[...] + b_ref[...]
  m = jnp.mean(s, axis=0, keepdims=True)
  v = jnp.mean((s - m) ** 2, axis=0, keepdims=True)
  h = (s - m) / jnp.sqrt(v + EPS) * g_ref[...] + beta_ref[...]
  o_ref[...] = _leaky(h)


def _bn_leaky(p0, p1, b, g, beta):
  return pl.pallas_call(
      _bn_leaky_body,
      out_shape=jax.ShapeDtypeStruct((N_NODES, 32), jnp.float32),
  )(p0, p1, b.reshape(1, 32), g.reshape(1, 32), beta.reshape(1, 32))


def _tail_body(p0_ref, p1_ref, w2_ref, b2_ref, g2_ref, beta2_ref,
               batch_ref, fc1w_ref, fc1b_ref, fc2w_ref, fc2b_ref,
               fc3w_ref, fc3b_ref, o_ref):
  s = p0_ref[...] + p1_ref[...]
  z = jnp.dot(s, w2_ref[...], preferred_element_type=jnp.float32)
  z = z + b2_ref[...]
  m = jnp.mean(z, axis=0, keepdims=True)
  v = jnp.mean((z - m) ** 2, axis=0, keepdims=True)
  h = _leaky((z - m) / jnp.sqrt(v + EPS) * g2_ref[...] + beta2_ref[...])
  gids = lax.broadcasted_iota(jnp.int32, (N_GRAPHS, N_NODES), 0)
  onehot = (gids == batch_ref[...]).astype(jnp.float32)
  pooled = jnp.dot(onehot, h, preferred_element_type=jnp.float32)
  o1 = _leaky(jnp.dot(pooled, fc1w_ref[...],
                      preferred_element_type=jnp.float32) + fc1b_ref[...])
  o2 = _leaky(jnp.dot(o1, fc2w_ref[...],
                      preferred_element_type=jnp.float32) + fc2b_ref[...])
  o3 = _leaky(jnp.dot(o2, fc3w_ref[...],
                      preferred_element_type=jnp.float32) + fc3b_ref[...])
  o_ref[...] = o3


def _tail(p0, p1, w2, b2, g2, beta2, batch, fc1_w, fc1_b, fc2_w, fc2_b,
          fc3_w, fc3_b):
  return pl.pallas_call(
      _tail_body,
      out_shape=jax.ShapeDtypeStruct((N_GRAPHS, 2), jnp.float32),
  )(p0, p1, w2, b2.reshape(1, 64), g2.reshape(1, 64),
    beta2.reshape(1, 64), batch.reshape(1, N_NODES),
    fc1_w.T, fc1_b.reshape(1, 32), fc2_w.T, fc2_b.reshape(1, 16),
    fc3_w.T, fc3_b.reshape(1, 2))


def kernel(x, edge_index, edge_weigth, batch, W1, b1, g1, beta1,
           W2, b2, g2, beta2, fc1_w, fc1_b, fc2_w, fc2_b, fc3_w, fc3_b):
  pad = E_PAD - E
  srcp = jnp.concatenate(
      [edge_index[0], jnp.zeros((pad,), jnp.int32)]).reshape(NW, NCH, CH)
  dstp = jnp.concatenate(
      [edge_index[1], jnp.zeros((pad,), jnp.int32)]).reshape(NW, NCH, CH)
  wp = jnp.concatenate(
      [edge_weigth, jnp.zeros((pad,), jnp.float32)]).reshape(NW, NCH, CH)
  zrows = jnp.zeros((ROWS_PER_TILE, 32), jnp.float32)

  h0 = _matmul_x_w1(x, W1)
  parts1 = _sc_aggregate(h0, srcp, dstp, wp, zrows)

  bnc1 = jnp.stack([b1, g1, beta1]).reshape(3, 2, 16)
  parts2 = _sc_aggregate(parts1, srcp, dstp, wp, zrows, bnc=bnc1)
  return _tail(parts2[0, :N_NODES], parts2[1, :N_NODES], W2, b2, g2,
               beta2, batch, fc1_w, fc1_b, fc2_w, fc2_b, fc3_w, fc3_b)
